# baseline (speedup 1.0000x reference)
import functools

import jax
import jax.numpy as jnp
from jax import lax
from jax.experimental import pallas as pl
from jax.experimental.pallas import tpu as pltpu
from jax.experimental.pallas import tpu_sc as plsc

E = 320000
N = 10000
D = 128
NC = 2
NS = 16
L = 16
NW = NC * NS
EPW = E // NW
K = 80
NCHUNK = EPW // K
G = K // L
DSTEP = 8
ZROWS_PER_TILE = 1000


def _sc_body(z_hbm, src_hbm, dst_hbm, out_hbm,
             zsp, si0, si1, di0, di1, sr0, dr0, sr1, dr1, ob0, ob1,
             isem0, isem1, rsem0, rsem1, osem0, osem1):
    sid = lax.axis_index("s")
    wid = sid * NC + lax.axis_index("c")
    base = wid * EPW

    @pl.when(sid < N // ZROWS_PER_TILE)
    def _():
        pltpu.sync_copy(
            z_hbm.at[pl.ds(sid * ZROWS_PER_TILE, ZROWS_PER_TILE)],
            zsp.at[pl.ds(sid * ZROWS_PER_TILE, ZROWS_PER_TILE)])

    plsc.subcore_barrier()

    sidx = (si0, si1)
    didx = (di0, di1)
    srows = (sr0, sr1)
    drows = (dr0, dr1)
    obufs = (ob0, ob1)
    isems = (isem0, isem1)
    rsems = (rsem0, rsem1)
    osems = (osem0, osem1)

    def issue_idx(c, b):
        pltpu.async_copy(src_hbm.at[pl.ds(base + c * K, K)], sidx[b], isems[b])
        pltpu.async_copy(dst_hbm.at[pl.ds(base + c * K, K)], didx[b], isems[b])

    def wait_idx(c, b):
        pltpu.make_async_copy(
            src_hbm.at[pl.ds(base + c * K, K)], sidx[b], isems[b]).wait()
        pltpu.make_async_copy(
            dst_hbm.at[pl.ds(base + c * K, K)], didx[b], isems[b]).wait()

    def issue_rows(b):
        pltpu.async_copy(zsp.at[sidx[b]], srows[b], rsems[b])
        pltpu.async_copy(zsp.at[didx[b]], drows[b], rsems[b])

    def wait_rows(b):
        pltpu.make_async_copy(zsp.at[sidx[b]], srows[b], rsems[b]).wait()
        pltpu.make_async_copy(zsp.at[didx[b]], drows[b], rsems[b]).wait()

    def issue_ostore(c, b):
        pltpu.async_copy(obufs[b], out_hbm.at[pl.ds(base + c * K, K)], osems[b])

    def wait_ostore(c, b):
        pltpu.make_async_copy(
            obufs[b], out_hbm.at[pl.ds(base + c * K, K)], osems[b]).wait()

    lanes = lax.iota(jnp.int32, L)

    def compute(b):
        sref = srows[b]
        dref = drows[b]
        for g in range(G):
            eids = lanes + (g * L)

            def dstep(t, acc):
                for dd in range(DSTEP):
                    d = t * DSTEP + dd
                    dvec = jnp.full((L,), d, dtype=jnp.int32)
                    sv = plsc.load_gather(sref, [eids, dvec])
                    dv = plsc.load_gather(dref, [eids, dvec])
                    acc = acc + sv * dv
                return acc

            acc = lax.fori_loop(0, D // DSTEP, dstep,
                                jnp.zeros((L,), jnp.float32))
            obufs[b][pl.ds(g * L, L)] = 1.0 / (1.0 + jnp.exp(-acc))

    def step(c, b):
        @pl.when(c + 1 < NCHUNK)
        def _():
            wait_idx(c + 1, 1 - b)
            issue_rows(1 - b)
        wait_rows(b)

        @pl.when(c + 2 < NCHUNK)
        def _():
            issue_idx(c + 2, b)

        @pl.when(c >= 2)
        def _():
            wait_ostore(c - 2, b)
        compute(b)
        issue_ostore(c, b)

    issue_idx(0, 0)
    issue_idx(1, 1)
    wait_idx(0, 0)
    issue_rows(0)

    def chunk_pair(i, carry):
        step(2 * i, 0)
        step(2 * i + 1, 1)
        return carry

    lax.fori_loop(0, NCHUNK // 2, chunk_pair, 0)
    step(NCHUNK - 1, 0)
    wait_ostore(NCHUNK - 2, 1)
    wait_ostore(NCHUNK - 1, 0)


@jax.jit
def _run(z, src, dst):
    mesh = plsc.VectorSubcoreMesh(
        core_axis_name="c", subcore_axis_name="s",
        num_cores=NC, num_subcores=NS)
    return pl.kernel(
        _sc_body,
        out_type=jax.ShapeDtypeStruct((E,), jnp.float32),
        mesh=mesh,
        compiler_params=pltpu.CompilerParams(needs_layout_passes=False),
        scratch_types=[
            pltpu.VMEM_SHARED((N, D), jnp.float32),
            pltpu.VMEM((K,), jnp.int32),
            pltpu.VMEM((K,), jnp.int32),
            pltpu.VMEM((K,), jnp.int32),
            pltpu.VMEM((K,), jnp.int32),
            pltpu.VMEM((K, D), jnp.float32),
            pltpu.VMEM((K, D), jnp.float32),
            pltpu.VMEM((K, D), jnp.float32),
            pltpu.VMEM((K, D), jnp.float32),
            pltpu.VMEM((K,), jnp.float32),
            pltpu.VMEM((K,), jnp.float32),
            pltpu.SemaphoreType.DMA,
            pltpu.SemaphoreType.DMA,
            pltpu.SemaphoreType.DMA,
            pltpu.SemaphoreType.DMA,
            pltpu.SemaphoreType.DMA,
            pltpu.SemaphoreType.DMA,
        ],
    )(z, src, dst)


def kernel(z, edge_index):
    ei = edge_index.astype(jnp.int32)
    return _run(z, ei[0], ei[1])


# speedup vs baseline: 6.6974x; 6.6974x over previous
import functools

import jax
import jax.numpy as jnp
from jax import lax
from jax.experimental import pallas as pl
from jax.experimental.pallas import tpu as pltpu
from jax.experimental.pallas import tpu_sc as plsc

E = 320000
N = 10000
D = 128
NC = 2
NS = 16
L = 16
NW = NC * NS
EPW = E // NW
K = 80
NCHUNK = EPW // K
G = K // L
DSTEP = 8
ZROWS_PER_TILE = 1000


def _sc_body(z_hbm, src_hbm, dst_hbm, out_hbm,
             zsp, si0, si1, di0, di1, sr0, dr0, sr1, dr1, ob0, ob1,
             isem0, isem1, rsem0, rsem1, osem0, osem1):
    sid = lax.axis_index("s")
    wid = sid * NC + lax.axis_index("c")
    base = wid * EPW

    @pl.when(sid < N // ZROWS_PER_TILE)
    def _():
        pltpu.sync_copy(
            z_hbm.at[pl.ds(sid * ZROWS_PER_TILE, ZROWS_PER_TILE)],
            zsp.at[pl.ds(sid * ZROWS_PER_TILE, ZROWS_PER_TILE)])

    plsc.subcore_barrier()

    sidx = (si0, si1)
    didx = (di0, di1)
    srows = (sr0, sr1)
    drows = (dr0, dr1)
    obufs = (ob0, ob1)
    isems = (isem0, isem1)
    rsems = (rsem0, rsem1)
    osems = (osem0, osem1)

    def issue_idx(c, b):
        pltpu.async_copy(src_hbm.at[pl.ds(base + c * K, K)], sidx[b], isems[b])
        pltpu.async_copy(dst_hbm.at[pl.ds(base + c * K, K)], didx[b], isems[b])

    def wait_idx(c, b):
        pltpu.make_async_copy(
            src_hbm.at[pl.ds(base + c * K, K)], sidx[b], isems[b]).wait()
        pltpu.make_async_copy(
            dst_hbm.at[pl.ds(base + c * K, K)], didx[b], isems[b]).wait()

    def issue_rows(b):
        pltpu.async_copy(zsp.at[sidx[b]], srows[b], rsems[b])
        pltpu.async_copy(zsp.at[didx[b]], drows[b], rsems[b])

    def wait_rows(b):
        pltpu.make_async_copy(zsp.at[sidx[b]], srows[b], rsems[b]).wait()
        pltpu.make_async_copy(zsp.at[didx[b]], drows[b], rsems[b]).wait()

    def issue_ostore(c, b):
        pltpu.async_copy(obufs[b], out_hbm.at[pl.ds(base + c * K, K)], osems[b])

    def wait_ostore(c, b):
        pltpu.make_async_copy(
            obufs[b], out_hbm.at[pl.ds(base + c * K, K)], osems[b]).wait()

    lanes = lax.iota(jnp.int32, L)

    def compute(b):
        sref = srows[b]
        dref = drows[b]
        for g in range(G):
            eids = lanes + (g * L)

            def dstep(t, acc):
                for dd in range(DSTEP):
                    dvec = (lanes + (t * DSTEP + dd)) & (D - 1)
                    sv = plsc.load_gather(sref, [eids, dvec])
                    dv = plsc.load_gather(dref, [eids, dvec])
                    acc = acc + sv * dv
                return acc

            acc = lax.fori_loop(0, D // DSTEP, dstep,
                                jnp.zeros((L,), jnp.float32))
            obufs[b][pl.ds(g * L, L)] = 1.0 / (1.0 + jnp.exp(-acc))

    def step(c, b):
        @pl.when(c + 1 < NCHUNK)
        def _():
            wait_idx(c + 1, 1 - b)
            issue_rows(1 - b)
        wait_rows(b)

        @pl.when(c + 2 < NCHUNK)
        def _():
            issue_idx(c + 2, b)

        @pl.when(c >= 2)
        def _():
            wait_ostore(c - 2, b)
        compute(b)
        issue_ostore(c, b)

    issue_idx(0, 0)
    issue_idx(1, 1)
    wait_idx(0, 0)
    issue_rows(0)

    def chunk_pair(i, carry):
        step(2 * i, 0)
        step(2 * i + 1, 1)
        return carry

    lax.fori_loop(0, NCHUNK // 2, chunk_pair, 0)
    step(NCHUNK - 1, 0)
    wait_ostore(NCHUNK - 2, 1)
    wait_ostore(NCHUNK - 1, 0)


@jax.jit
def _run(z, src, dst):
    mesh = plsc.VectorSubcoreMesh(
        core_axis_name="c", subcore_axis_name="s",
        num_cores=NC, num_subcores=NS)
    return pl.kernel(
        _sc_body,
        out_type=jax.ShapeDtypeStruct((E,), jnp.float32),
        mesh=mesh,
        compiler_params=pltpu.CompilerParams(needs_layout_passes=False),
        scratch_types=[
            pltpu.VMEM_SHARED((N, D), jnp.float32),
            pltpu.VMEM((K,), jnp.int32),
            pltpu.VMEM((K,), jnp.int32),
            pltpu.VMEM((K,), jnp.int32),
            pltpu.VMEM((K,), jnp.int32),
            pltpu.VMEM((K, D), jnp.float32),
            pltpu.VMEM((K, D), jnp.float32),
            pltpu.VMEM((K, D), jnp.float32),
            pltpu.VMEM((K, D), jnp.float32),
            pltpu.VMEM((K,), jnp.float32),
            pltpu.VMEM((K,), jnp.float32),
            pltpu.SemaphoreType.DMA,
            pltpu.SemaphoreType.DMA,
            pltpu.SemaphoreType.DMA,
            pltpu.SemaphoreType.DMA,
            pltpu.SemaphoreType.DMA,
            pltpu.SemaphoreType.DMA,
        ],
    )(z, src, dst)


def kernel(z, edge_index):
    ei = edge_index.astype(jnp.int32)
    return _run(z, ei[0], ei[1])
